# Initial kernel scaffold; baseline (speedup 1.0000x reference)
import jax
import jax.numpy as jnp
from jax.experimental import pallas as pl


def kernel(z, W):
    raise NotImplementedError("write your pallas kernel here")


# speedup vs baseline: 1.4256x; 1.4256x over previous
import functools

import jax
import jax.numpy as jnp
from jax import lax
from jax.experimental import pallas as pl
from jax.experimental.pallas import tpu as pltpu
from jax.experimental.pallas import tpu_sc as plsc

def _argmin_codes(z3, W, PC=512, CC=1024, interpret=False):
    B, C, S = z3.shape
    K, _ = W.shape
    NPC = S // PC
    NCC = K // CC

    def body(w_ref, z_ref, idx_ref):
        zr = z_ref[0]
        zsq = jnp.sum(zr * zr, axis=0, keepdims=True)

        def step(c, carry):
            rmin, ridx = carry
            wc = w_ref[pl.ds(c * CC, CC), :]
            wsq = jnp.sum(wc * wc, axis=1, keepdims=True)
            mm = lax.dot_general(wc, zr, (((1,), (0,)), ((), ())),
                                 preferred_element_type=jnp.float32)
            d = (zsq + wsq) - 2.0 * mm
            dmin = jnp.min(d, axis=0, keepdims=True)
            bi = lax.broadcasted_iota(jnp.int32, (CC, PC), 0) + c * CC
            cidx = jnp.min(jnp.where(d == dmin, bi, 2**30), axis=0,
                           keepdims=True)
            upd = dmin < rmin
            return jnp.where(upd, dmin, rmin), jnp.where(upd, cidx, ridx)

        rmin0 = jnp.full((1, PC), jnp.inf, jnp.float32)
        ridx0 = jnp.zeros((1, PC), jnp.int32)
        _, ridx = lax.fori_loop(0, NCC, step, (rmin0, ridx0))
        idx_ref[0, 0] = ridx

    return pl.pallas_call(
        body,
        grid=(B, NPC),
        in_specs=[
            pl.BlockSpec((K, C), lambda b, p: (0, 0)),
            pl.BlockSpec((1, C, PC), lambda b, p: (b, 0, p)),
        ],
        out_specs=pl.BlockSpec((1, 1, 1, PC), lambda b, p: (b, p, 0, 0)),
        out_shape=jax.ShapeDtypeStruct((B, NPC, 1, PC), jnp.int32),
        interpret=interpret,
    )(W, z3)


def _sc_lookup(WT, idx2, z3):
    B, C, S = z3.shape
    mesh = plsc.VectorSubcoreMesh(core_axis_name="c", subcore_axis_name="s")

    @functools.partial(
        pl.kernel, mesh=mesh,
        out_type=jax.ShapeDtypeStruct((B, C, S), jnp.float32),
        compiler_params=pltpu.CompilerParams(needs_layout_passes=False),
        scratch_types=[
            pltpu.VMEM((B, S), jnp.int32),
            pltpu.VMEM((WT.shape[1],), jnp.float32),
            pltpu.VMEM((B, S), jnp.float32),
            pltpu.VMEM((B, S), jnp.float32),
        ],
    )
    def body(wt_hbm, idx_hbm, z_hbm, out_hbm, idx_v, wt_v, z_v, out_v):
        ch = lax.axis_index("s") * 2 + lax.axis_index("c")
        pltpu.sync_copy(idx_hbm, idx_v)
        pltpu.sync_copy(wt_hbm.at[ch], wt_v)
        for b in range(B):
            pltpu.sync_copy(z_hbm.at[b, ch], z_v.at[b])

        def step(i, carry):
            s = i * 16
            for b in range(B):
                iv = idx_v[b, pl.ds(s, 16)]
                g = plsc.load_gather(wt_v, [iv])
                zv = z_v[b, pl.ds(s, 16)]
                out_v[b, pl.ds(s, 16)] = zv + (g - zv)
            return carry

        lax.fori_loop(0, S // 16, step, 0)
        for b in range(B):
            pltpu.sync_copy(out_v.at[b], out_hbm.at[b, ch])

    return body(WT, idx2, z3)


def kernel(z, W):
    B, C, T, H, Wd = z.shape
    S = T * H * Wd
    z3 = z.reshape(B, C, S)
    idx4 = _argmin_codes(z3, W)
    idx2 = idx4.reshape(B, S)
    zq3 = _sc_lookup(jnp.transpose(W), idx2, z3)
    return zq3.reshape(B, C, T, H, Wd), idx2.reshape(B, T, H, Wd)


# speedup vs baseline: 1.5207x; 1.0667x over previous
import functools

import jax
import jax.numpy as jnp
from jax import lax
from jax.experimental import pallas as pl
from jax.experimental.pallas import tpu as pltpu
from jax.experimental.pallas import tpu_sc as plsc

def _argmin_codes(z3, W, PC=512, CC=1024, interpret=False):
    B, C, S = z3.shape
    K, _ = W.shape
    NPC = S // PC
    NCC = K // CC

    def body(w_ref, z_ref, idx_ref):
        zr = z_ref[0]
        zsq = jnp.sum(zr * zr, axis=0, keepdims=True)

        bi = lax.broadcasted_iota(jnp.int32, (CC, 1), 0).astype(jnp.float32)

        def step(c, carry):
            run_d, run_j = carry
            wc = w_ref[pl.ds(c * CC, CC), :]
            wsq = jnp.sum(wc * wc, axis=1, keepdims=True)
            mm2 = lax.dot_general(wc + wc, zr, (((1,), (0,)), ((), ())),
                                  preferred_element_type=jnp.float32)
            d = (zsq + wsq) - mm2
            dmin = jnp.min(d, axis=0, keepdims=True)
            jrel = jnp.min(jnp.where(d == dmin, bi, 3e38), axis=0,
                           keepdims=True)
            j = jrel.astype(jnp.int32) + c * CC
            upd = dmin < run_d
            return jnp.where(upd, dmin, run_d), jnp.where(upd, j, run_j)

        rd0 = jnp.full((1, PC), jnp.inf, jnp.float32)
        rj0 = jnp.zeros((1, PC), jnp.int32)
        _, rj = lax.fori_loop(0, NCC, step, (rd0, rj0))
        idx_ref[0, 0] = rj

    return pl.pallas_call(
        body,
        grid=(B, NPC),
        in_specs=[
            pl.BlockSpec((K, C), lambda b, p: (0, 0)),
            pl.BlockSpec((1, C, PC), lambda b, p: (b, 0, p)),
        ],
        out_specs=pl.BlockSpec((1, 1, 1, PC), lambda b, p: (b, p, 0, 0)),
        out_shape=jax.ShapeDtypeStruct((B, NPC, 1, PC), jnp.int32),
        interpret=interpret,
    )(W, z3)


def _sc_lookup(WT, idx2, z3):
    B, C, S = z3.shape
    mesh = plsc.VectorSubcoreMesh(core_axis_name="c", subcore_axis_name="s")

    @functools.partial(
        pl.kernel, mesh=mesh,
        out_type=jax.ShapeDtypeStruct((B, C, S), jnp.float32),
        compiler_params=pltpu.CompilerParams(needs_layout_passes=False),
        scratch_types=[
            pltpu.VMEM((B, S), jnp.int32),
            pltpu.VMEM((WT.shape[1],), jnp.float32),
            pltpu.VMEM((B, S), jnp.float32),
            pltpu.VMEM((B, S), jnp.float32),
        ],
    )
    def body(wt_hbm, idx_hbm, z_hbm, out_hbm, idx_v, wt_v, z_v, out_v):
        ch = lax.axis_index("s") * 2 + lax.axis_index("c")
        pltpu.sync_copy(idx_hbm, idx_v)
        pltpu.sync_copy(wt_hbm.at[ch], wt_v)
        for b in range(B):
            pltpu.sync_copy(z_hbm.at[b, ch], z_v.at[b])

        def step(i, carry):
            s = i * 16
            for b in range(B):
                iv = idx_v[b, pl.ds(s, 16)]
                g = plsc.load_gather(wt_v, [iv])
                zv = z_v[b, pl.ds(s, 16)]
                out_v[b, pl.ds(s, 16)] = zv + (g - zv)
            return carry

        lax.fori_loop(0, S // 16, step, 0)
        for b in range(B):
            pltpu.sync_copy(out_v.at[b], out_hbm.at[b, ch])

    return body(WT, idx2, z3)


def kernel(z, W):
    B, C, T, H, Wd = z.shape
    S = T * H * Wd
    z3 = z.reshape(B, C, S)
    idx4 = _argmin_codes(z3, W)
    idx2 = idx4.reshape(B, S)
    zq3 = _sc_lookup(jnp.transpose(W), idx2, z3)
    return zq3.reshape(B, C, T, H, Wd), idx2.reshape(B, T, H, Wd)
